# baseline (speedup 1.0000x reference)
import functools

import jax
import jax.numpy as jnp
from jax import lax
from jax.experimental import pallas as pl
from jax.experimental.pallas import tpu as pltpu
from jax.experimental.pallas import tpu_sc as plsc

VOCAB = 1000
MAIN = 896
TAILW = 128
TAIL = VOCAB - MAIN
N_EMBD = 32
B, T = 1024, 200
ROWS = B * T

NC, NS = 2, 16
NW = NC * NS
RPW = ROWS // NW
CH = 32
NCH = RPW // CH


def _bigram_matrix_body(tab_ref, wm_ref, bm_ref, wt_ref, bt_ref, mm_ref, mt_ref):
    tab = tab_ref[...]
    mm_ref[...] = (
        jnp.dot(tab, wm_ref[...], preferred_element_type=jnp.float32) + bm_ref[...]
    )
    mt_ref[...] = (
        jnp.dot(tab, wt_ref[...], preferred_element_type=jnp.float32) + bt_ref[...]
    )


def _compute_m(table, w, b):
    return pl.pallas_call(
        _bigram_matrix_body,
        out_shape=(
            jax.ShapeDtypeStruct((VOCAB, MAIN), jnp.float32),
            jax.ShapeDtypeStruct((VOCAB, TAILW), jnp.float32),
        ),
    )(
        table,
        w[:, :MAIN],
        b[:MAIN].reshape(1, MAIN),
        w[:, VOCAB - TAILW :],
        b[VOCAB - TAILW :].reshape(1, TAILW),
    )


def _gather_rows_body(mm_hbm, mt_hbm, idx_hbm, out_hbm,
                      idx_v, bufo_a, bufo_b, buft_a, buft_b,
                      semm_a, semm_b, semt_a, semt_b):
    wid = lax.axis_index("s") * NC + lax.axis_index("c")
    base = wid * RPW
    pltpu.sync_copy(idx_hbm.at[wid], idx_v)

    def issue(j, bufo, buft, semm, semt):
        pltpu.async_copy(mm_hbm.at[idx_v.at[j]], bufo.at[:, pl.ds(0, MAIN)], semm)
        pltpu.async_copy(mt_hbm.at[idx_v.at[j]], buft.at[pl.ds(0, CH)], semt)

    zero = lax.axis_index("c") * 0

    def drain(j, bufo, buft, semm, semt):
        pltpu.make_async_copy(
            mm_hbm.at[idx_v.at[j]], bufo.at[:, pl.ds(0, MAIN)], semm
        ).wait()
        pltpu.make_async_copy(
            mt_hbm.at[idx_v.at[j]], buft.at[pl.ds(0, CH)], semt
        ).wait()
        for r in range(CH):
            for o in (0, 16, 32, 48, 64, 80):
                bufo[r, pl.ds(MAIN + o, 16)] = buft[r, pl.ds(TAILW - TAIL + o, 16)]
            bufo[r, pl.ds(zero + 992, 16)] = buft[r, pl.ds(zero + 120, 16)]
        pltpu.sync_copy(bufo, out_hbm.at[pl.ds(base + j * CH, CH)])

    issue(0, bufo_a, buft_a, semm_a, semt_a)

    def step(i, _):
        j = 2 * i
        issue(j + 1, bufo_b, buft_b, semm_b, semt_b)
        drain(j, bufo_a, buft_a, semm_a, semt_a)
        issue(j + 2, bufo_a, buft_a, semm_a, semt_a)
        drain(j + 1, bufo_b, buft_b, semm_b, semt_b)
        return 0

    lax.fori_loop(0, NCH // 2 - 1, step, 0, unroll=False)

    j = NCH - 2
    issue(j + 1, bufo_b, buft_b, semm_b, semt_b)
    drain(j, bufo_a, buft_a, semm_a, semt_a)
    drain(j + 1, bufo_b, buft_b, semm_b, semt_b)


@functools.lru_cache(maxsize=1)
def _make_gather_rows():
    mesh = plsc.VectorSubcoreMesh(core_axis_name="c", subcore_axis_name="s")
    return pl.kernel(
        _gather_rows_body,
        mesh=mesh,
        out_type=jax.ShapeDtypeStruct((ROWS, VOCAB), jnp.float32),
        scratch_types=[
            pltpu.VMEM((NCH, CH), jnp.int32),
            pltpu.VMEM((CH, VOCAB), jnp.float32),
            pltpu.VMEM((CH, VOCAB), jnp.float32),
            pltpu.VMEM((CH + 8, TAILW), jnp.float32),
            pltpu.VMEM((CH + 8, TAILW), jnp.float32),
            pltpu.SemaphoreType.DMA,
            pltpu.SemaphoreType.DMA,
            pltpu.SemaphoreType.DMA,
            pltpu.SemaphoreType.DMA,
        ],
    )


def kernel(idx, token_embedding_table, lm_head_w, lm_head_b):
    m_main, m_tail = _compute_m(token_embedding_table, lm_head_w, lm_head_b)
    idx_chunks = idx.reshape(NW, NCH, CH)
    out = _make_gather_rows()(m_main, m_tail, idx_chunks)
    return out.reshape(B, T, VOCAB)


# speedup vs baseline: 1.0003x; 1.0003x over previous
import functools

import jax
import jax.numpy as jnp
from jax import lax
from jax.experimental import pallas as pl
from jax.experimental.pallas import tpu as pltpu
from jax.experimental.pallas import tpu_sc as plsc

VOCAB = 1000
MAIN = 896
TAILW = 128
TAIL = VOCAB - MAIN
N_EMBD = 32
B, T = 1024, 200
ROWS = B * T

NC, NS = 2, 16
NW = NC * NS
RPW = ROWS // NW
CH = 32
NCH = RPW // CH


def _bigram_matrix_body(tab_ref, wm_ref, bm_ref, wt_ref, bt_ref, mm_ref, mt_ref):
    tab = tab_ref[...]
    mm_ref[...] = (
        jnp.dot(tab, wm_ref[...], preferred_element_type=jnp.float32) + bm_ref[...]
    )
    mt_ref[...] = (
        jnp.dot(tab, wt_ref[...], preferred_element_type=jnp.float32) + bt_ref[...]
    )


def _compute_m(table, w, b):
    return pl.pallas_call(
        _bigram_matrix_body,
        out_shape=(
            jax.ShapeDtypeStruct((VOCAB, MAIN), jnp.float32),
            jax.ShapeDtypeStruct((VOCAB, TAILW), jnp.float32),
        ),
    )(
        table,
        w[:, :MAIN],
        b[:MAIN].reshape(1, MAIN),
        w[:, VOCAB - TAILW :],
        b[VOCAB - TAILW :].reshape(1, TAILW),
    )


NB = 3


def _gather_rows_body(mm_hbm, mt_hbm, idx_hbm, out_hbm,
                      idxb0, idxb1, idxb2, bufo0, bufo1, bufo2,
                      buft0, buft1, buft2,
                      semi0, semi1, semi2, semm0, semm1, semm2,
                      semt0, semt1, semt2, semw0, semw1, semw2):
    idxb = (idxb0, idxb1, idxb2)
    bufo = (bufo0, bufo1, bufo2)
    buft = (buft0, buft1, buft2)
    semi = (semi0, semi1, semi2)
    semm = (semm0, semm1, semm2)
    semt = (semt0, semt1, semt2)
    semw = (semw0, semw1, semw2)

    wid = lax.axis_index("s") * NC + lax.axis_index("c")
    base = wid * RPW

    def issue_idx(j, k):
        pltpu.async_copy(idx_hbm.at[wid, j], idxb[k], semi[k])

    def wait_idx(j, k):
        pltpu.make_async_copy(idx_hbm.at[wid, j], idxb[k], semi[k]).wait()

    def issue_gather(k):
        pltpu.async_copy(mm_hbm.at[idxb[k]], bufo[k].at[:, pl.ds(0, MAIN)],
                         semm[k])
        pltpu.async_copy(mt_hbm.at[idxb[k]], buft[k].at[pl.ds(0, CH)], semt[k])

    def wait_gather(k):
        pltpu.make_async_copy(
            mm_hbm.at[idxb[k]], bufo[k].at[:, pl.ds(0, MAIN)], semm[k]
        ).wait()
        pltpu.make_async_copy(
            mt_hbm.at[idxb[k]], buft[k].at[pl.ds(0, CH)], semt[k]
        ).wait()

    def wait_write(j, k):
        pltpu.make_async_copy(
            bufo[k], out_hbm.at[pl.ds(base + j * CH, CH)], semw[k]
        ).wait()

    zero = lax.axis_index("c") * 0

    def visit(j, k):
        kn1 = (k + 1) % NB
        kn2 = (k + 2) % NB
        @pl.when(j + 1 < NCH)
        def _():
            wait_idx(j + 1, kn1)
            issue_gather(kn1)

        wait_gather(k)
        for r in range(CH):
            for o in (0, 16, 32, 48, 64, 80):
                bufo[k][r, pl.ds(MAIN + o, 16)] = \
                    buft[k][r, pl.ds(TAILW - TAIL + o, 16)]
            bufo[k][r, pl.ds(zero + 992, 16)] = buft[k][r, pl.ds(zero + 120, 16)]
        pltpu.async_copy(bufo[k], out_hbm.at[pl.ds(base + j * CH, CH)], semw[k])
        @pl.when(j >= 1)
        def _():
            wait_write(j - 1, kn2)

        @pl.when(j + 2 < NCH)
        def _():
            issue_idx(j + 2, kn2)

    issue_idx(0, 0)
    wait_idx(0, 0)
    issue_gather(0)
    issue_idx(1, 1)

    def group(g, _):
        for k in range(NB):
            visit(NB * g + k, k)
        return 0

    lax.fori_loop(0, (NCH - 2) // NB, group, 0, unroll=False)

    visit(jnp.int32(NCH - 2), 0)
    visit(jnp.int32(NCH - 1), 1)
    wait_write(NCH - 1, 1)


@functools.lru_cache(maxsize=1)
def _make_gather_rows():
    mesh = plsc.VectorSubcoreMesh(core_axis_name="c", subcore_axis_name="s")
    return pl.kernel(
        _gather_rows_body,
        mesh=mesh,
        out_type=jax.ShapeDtypeStruct((ROWS, VOCAB), jnp.float32),
        scratch_types=(
            [pltpu.VMEM((CH,), jnp.int32) for _ in range(NB)]
            + [pltpu.VMEM((CH, VOCAB), jnp.float32) for _ in range(NB)]
            + [pltpu.VMEM((CH + 8, TAILW), jnp.float32) for _ in range(NB)]
            + [pltpu.SemaphoreType.DMA for _ in range(4 * NB)]
        ),
    )


def kernel(idx, token_embedding_table, lm_head_w, lm_head_b):
    m_main, m_tail = _compute_m(token_embedding_table, lm_head_w, lm_head_b)
    idx_chunks = idx.reshape(NW, NCH, CH)
    out = _make_gather_rows()(m_main, m_tail, idx_chunks)
    return out.reshape(B, T, VOCAB)
